# baseline (speedup 1.0000x reference)
import functools

import jax
import jax.numpy as jnp
from jax import lax
from jax.experimental import pallas as pl
from jax.experimental.pallas import tpu as pltpu
from jax.experimental.pallas import tpu_sc as plsc

N = 500000
D = 64
NC = 2
NS = 16
NW = NC * NS

C = 128
NFULL = N // C
TAIL = N - NFULL * C
TAIL_BASE = NFULL * C


def _gather_body(x_hbm, idx_hbm, out_hbm, idx_v, rows_v, idx_t, rows_t, sem):
    wid = lax.axis_index("s") * NC + lax.axis_index("c")
    n_w = (NFULL - 1 - wid) // NW + 1

    def body(i, carry):
        base = (wid + i * NW) * C
        pltpu.sync_copy(idx_hbm.at[pl.ds(base, C)], idx_v)
        pltpu.async_copy(x_hbm.at[idx_v], rows_v, sem).wait()
        pltpu.sync_copy(rows_v, out_hbm.at[pl.ds(base, C)])
        return carry

    lax.fori_loop(0, n_w, body, 0)

    @pl.when(wid == NW - 1)
    def _tail():
        pltpu.sync_copy(idx_hbm.at[pl.ds(TAIL_BASE, TAIL)], idx_t)
        pltpu.async_copy(x_hbm.at[idx_t], rows_t, sem).wait()
        pltpu.sync_copy(rows_t, out_hbm.at[pl.ds(TAIL_BASE, TAIL)])


_gather = functools.partial(
    pl.kernel,
    out_type=jax.ShapeDtypeStruct((N, D), jnp.float32),
    mesh=plsc.VectorSubcoreMesh(core_axis_name="c", subcore_axis_name="s"),
    scratch_types=[
        pltpu.VMEM((C,), jnp.int32),
        pltpu.VMEM((C, D), jnp.float32),
        pltpu.VMEM((TAIL,), jnp.int32),
        pltpu.VMEM((TAIL, D), jnp.float32),
        pltpu.SemaphoreType.DMA,
    ],
    compiler_params=pltpu.CompilerParams(use_tc_tiling_on_sc=False),
)(_gather_body)


@jax.jit
def kernel(x, indexer):
    return _gather(x, indexer.astype(jnp.int32))


# speedup vs baseline: 1.1455x; 1.1455x over previous
import functools

import jax
import jax.numpy as jnp
from jax import lax
from jax.experimental import pallas as pl
from jax.experimental.pallas import tpu as pltpu
from jax.experimental.pallas import tpu_sc as plsc

N = 500000
D = 64
NC = 2
NS = 16
NW = NC * NS

C = 800
NCHUNK = N // C
KMIN = NCHUNK // NW
NEXTRA = NCHUNK - KMIN * NW
MAXK = KMIN + 1


def _gather_body(x_hbm, idx_hbm, out_hbm,
                 idx_v, rows_v, gsem0, gsem1, osem0, osem1):
    wid = lax.axis_index("s") * NC + lax.axis_index("c")
    gsem = (gsem0, gsem1)
    osem = (osem0, osem1)

    def chunk_base(k):
        return (wid + k * NW) * C

    def wait_out(p):
        pltpu.make_async_copy(
            rows_v.at[p], out_hbm.at[pl.ds(0, C)], osem[p]).wait()

    def stage_load(k, p):
        base = chunk_base(k)
        pltpu.sync_copy(idx_hbm.at[pl.ds(base, C)], idx_v.at[p])
        pltpu.async_copy(x_hbm.at[idx_v.at[p]], rows_v.at[p], gsem[p])

    def stage_drain(k, p):
        pltpu.make_async_copy(
            x_hbm.at[idx_v.at[p]], rows_v.at[p], gsem[p]).wait()
        pltpu.async_copy(
            rows_v.at[p], out_hbm.at[pl.ds(chunk_base(k), C)], osem[p])

    for k in range(MAXK):
        p = k & 1
        if k < KMIN:
            if k >= 2:
                wait_out(p)
            stage_load(k, p)
        else:
            @pl.when(wid < NEXTRA)
            def _extra_load(k=k, p=p):
                wait_out(p)
                stage_load(k, p)
        if k >= 1:
            stage_drain(k - 1, 1 - p)

    @pl.when(wid < NEXTRA)
    def _extra_drain():
        stage_drain(KMIN, KMIN & 1)

    for p in range(2):
        wait_out(p)


_gather = functools.partial(
    pl.kernel,
    out_type=jax.ShapeDtypeStruct((N, D), jnp.float32),
    mesh=plsc.VectorSubcoreMesh(core_axis_name="c", subcore_axis_name="s"),
    scratch_types=[
        pltpu.VMEM((2, C), jnp.int32),
        pltpu.VMEM((2, C, D), jnp.float32),
        pltpu.SemaphoreType.DMA,
        pltpu.SemaphoreType.DMA,
        pltpu.SemaphoreType.DMA,
        pltpu.SemaphoreType.DMA,
    ],
    compiler_params=pltpu.CompilerParams(use_tc_tiling_on_sc=False),
)(_gather_body)


@jax.jit
def kernel(x, indexer):
    return _gather(x, indexer.astype(jnp.int32))
